# baseline (speedup 1.0000x reference)
import functools

import jax
import jax.numpy as jnp
from jax import lax
from jax.experimental import pallas as pl
from jax.experimental.pallas import tpu as pltpu
from jax.experimental.pallas import tpu_sc as plsc

N = 10000
NPAD = 10240
E = 160000
F_IN = 128
H = 8
C = 64
HC = H * C
NP = 4
NCORES = 2
NSUB = 16
LANES = 16

BN = 1024

K1 = 80
NCH1 = (E // NSUB) // K1
K2 = 40
NCH2 = (E // (NCORES * NSUB)) // K2

ROWS_PER_TILE = NPAD // NSUB
ZROWS = 64


def _tc_feats_body(x_ref, w1_ref, asrc_ref, adst_ref, h1t_ref, es_ref, ed_ref):
  h = jnp.dot(x_ref[...], w1_ref[...], preferred_element_type=jnp.float32)
  es_ref[...] = jnp.dot(h, asrc_ref[...], preferred_element_type=jnp.float32)
  ed_ref[...] = jnp.dot(h, adst_ref[...], preferred_element_type=jnp.float32)
  for p in range(NP):
    h1t_ref[p] = h[:, 128 * p:128 * (p + 1)]


def _tc_feats(x, w1, asrc, adst):
  grid = (NPAD // BN,)
  return pl.pallas_call(
      _tc_feats_body,
      grid=grid,
      in_specs=[
          pl.BlockSpec((BN, F_IN), lambda i: (i, 0)),
          pl.BlockSpec((F_IN, HC), lambda i: (0, 0)),
          pl.BlockSpec((HC, 16), lambda i: (0, 0)),
          pl.BlockSpec((HC, 16), lambda i: (0, 0)),
      ],
      out_specs=[
          pl.BlockSpec((NP, BN, 128), lambda i: (0, i, 0)),
          pl.BlockSpec((BN, 16), lambda i: (i, 0)),
          pl.BlockSpec((BN, 16), lambda i: (i, 0)),
      ],
      out_shape=[
          jax.ShapeDtypeStruct((NP, NPAD, 128), jnp.float32),
          jax.ShapeDtypeStruct((NPAD, 16), jnp.float32),
          jax.ShapeDtypeStruct((NPAD, 16), jnp.float32),
      ],
  )(x, w1, asrc, adst)


def _lane_take(vec, idx):
  return lax.gather(
      vec, idx[:, None],
      dimension_numbers=lax.GatherDimensionNumbers(
          offset_dims=(), collapsed_slice_dims=(0,), start_index_map=(0,)),
      slice_sizes=(1,),
      mode=lax.GatherScatterMode.PROMISE_IN_BOUNDS)


def _sc_l1_body(h1f, es_t, ed_t, src_h, dst_h,
                acc_out, den_out,
                acc_s, den_s, srcb, srcb2, dstb, g1, g2, rows, exb, zb, zbd,
                gsem, ssem, dsem):
  c = lax.axis_index("c")
  s = lax.axis_index("s")
  row0 = s * ROWS_PER_TILE

  def _zero_vec(ref, nrows, ncols):
    def zrow(i, _):
      for t in range(ncols // LANES):
        ref[i, pl.ds(t * LANES, LANES)] = jnp.zeros((LANES,), jnp.float32)
      return 0
    lax.fori_loop(0, nrows, zrow, 0)

  _zero_vec(zb, ZROWS, 128)
  _zero_vec(zbd, ZROWS, 16)

  def zero_acc():
    for r in range(ROWS_PER_TILE // ZROWS):
      pltpu.sync_copy(zb, acc_s.at[pl.ds(row0 + r * ZROWS, ZROWS)])

  zero_acc()
  for r in range(ROWS_PER_TILE // ZROWS):
    pltpu.sync_copy(zbd, den_s.at[pl.ds(row0 + r * ZROWS, ZROWS)])
  plsc.subcore_barrier()

  ept = E // NSUB

  def issue(i, nb, poff):
    off = pl.multiple_of(s * ept + i * K1, 8)
    pltpu.sync_copy(src_h.at[pl.ds(off, K1)], srcb.at[nb])
    pltpu.sync_copy(dst_h.at[pl.ds(off, K1)], dstb.at[nb])
    for t in range(K1 // LANES):
      srcb2[nb, pl.ds(t * LANES, LANES)] = (
          srcb[nb, pl.ds(t * LANES, LANES)] + poff)
    pltpu.async_copy(es_t.at[srcb.at[nb]], g1.at[nb], gsem.at[nb])
    pltpu.async_copy(ed_t.at[dstb.at[nb]], g2.at[nb], gsem.at[nb])
    pltpu.async_copy(h1f.at[srcb2.at[nb]], rows.at[nb], gsem.at[nb])

  def drain_g12(ib):
    pltpu.make_async_copy(es_t.at[srcb.at[ib]], g1.at[ib], gsem.at[ib]).wait()
    pltpu.make_async_copy(ed_t.at[dstb.at[ib]], g2.at[ib], gsem.at[ib]).wait()

  def drain_rows(ib):
    pltpu.make_async_copy(h1f.at[srcb2.at[ib]], rows.at[ib],
                          gsem.at[ib]).wait()

  def drain_scat(ib, q):
    pltpu.make_async_copy(rows.at[ib], acc_s.at[dstb.at[ib]],
                          ssem.at[ib]).wait()
    if q == 0:
      pltpu.make_async_copy(exb.at[ib], den_s.at[dstb.at[ib]],
                            dsem.at[ib]).wait()

  for q in range(2):
    p = 2 * c + q
    hA = jnp.full((LANES,), 2 * p, jnp.int32)
    hB = jnp.full((LANES,), 2 * p + 1, jnp.int32)
    poff = p * NPAD

    issue(0, 0, poff)

    def chunk(i, _):
      ib = lax.rem(i, 2)
      nb = 1 - ib

      @pl.when(i >= 1)
      def _():
        drain_scat(nb, q)

      @pl.when(i + 1 < NCH1)
      def _():
        issue(i + 1, nb, poff)

      drain_g12(ib)

      @plsc.parallel_loop(0, K1, unroll=8)
      def _(k):
        e = g1[ib, k] + g2[ib, k]
        exb[ib, k] = jnp.exp(jnp.maximum(e, 0.2 * e))

      drain_rows(ib)

      @plsc.parallel_loop(0, K1, unroll=4)
      def _(k):
        ex = exb[ib, k]
        bA = _lane_take(ex, hA)
        bB = _lane_take(ex, hB)
        for j in range(8):
          b = bA if j < 4 else bB
          rows[ib, k, pl.ds(16 * j, LANES)] = (
              rows[ib, k, pl.ds(16 * j, LANES)] * b)
      pltpu.async_copy(rows.at[ib], acc_s.at[dstb.at[ib]], ssem.at[ib],
                       add=True)
      if q == 0:
        pltpu.async_copy(exb.at[ib], den_s.at[dstb.at[ib]], dsem.at[ib],
                         add=True)
      return 0

    lax.fori_loop(0, NCH1, chunk, 0)
    drain_scat((NCH1 - 1) % 2, q)
    plsc.subcore_barrier()
    pltpu.sync_copy(acc_s.at[pl.ds(row0, ROWS_PER_TILE)],
                    acc_out.at[p, pl.ds(row0, ROWS_PER_TILE)])
    if q == 0:
      @pl.when(c == 0)
      def _():
        pltpu.sync_copy(den_s.at[pl.ds(row0, ROWS_PER_TILE)],
                        den_out.at[pl.ds(row0, ROWS_PER_TILE)])
      zero_acc()
      plsc.subcore_barrier()


def _sc_l1(h1f, es_t, ed_t, src, dst):
  mesh = plsc.VectorSubcoreMesh(core_axis_name="c", subcore_axis_name="s",
                                num_cores=NCORES, num_subcores=NSUB)
  f = pl.kernel(
      _sc_l1_body,
      compiler_params=pltpu.CompilerParams(use_tc_tiling_on_sc=False),
      out_type=[
          jax.ShapeDtypeStruct((NP, NPAD, 128), jnp.float32),
          jax.ShapeDtypeStruct((NPAD, 16), jnp.float32),
      ],
      mesh=mesh,
      scratch_types=[
          pltpu.VMEM_SHARED((NPAD, 128), jnp.float32),
          pltpu.VMEM_SHARED((NPAD, 16), jnp.float32),
          pltpu.VMEM((2, K1), jnp.int32),
          pltpu.VMEM((2, K1), jnp.int32),
          pltpu.VMEM((2, K1), jnp.int32),
          pltpu.VMEM((2, K1, 16), jnp.float32),
          pltpu.VMEM((2, K1, 16), jnp.float32),
          pltpu.VMEM((2, K1, 128), jnp.float32),
          pltpu.VMEM((2, K1, 16), jnp.float32),
          pltpu.VMEM((ZROWS, 128), jnp.float32),
          pltpu.VMEM((ZROWS, 16), jnp.float32),
          pltpu.SemaphoreType.DMA((2,)),
          pltpu.SemaphoreType.DMA((2,)),
          pltpu.SemaphoreType.DMA((2,)),
      ],
  )
  return f(h1f, es_t, ed_t, src, dst)


def _tc_mid_body(acc_ref, den_ref, b1_ref, w2_ref, as2_ref, ad2_ref,
                 ta_ref, tb_ref, tc_ref):
  den = den_ref[...]
  bn = den.shape[0]
  h2 = jnp.zeros((bn, 128), jnp.float32)
  for p in range(NP):
    dA = jnp.broadcast_to(den[:, 2 * p:2 * p + 1], (bn, 64))
    dB = jnp.broadcast_to(den[:, 2 * p + 1:2 * p + 2], (bn, 64))
    d128 = jnp.concatenate([dA, dB], axis=1)
    hp = acc_ref[p] / (d128 + 1e-16) + b1_ref[:, 128 * p:128 * (p + 1)]
    hp = jnp.where(hp > 0, hp, jnp.exp(hp) - 1.0)
    h2 = h2 + jnp.dot(hp, w2_ref[pl.ds(128 * p, 128), :],
                      preferred_element_type=jnp.float32)
  es2 = jnp.sum(h2 * as2_ref[...], axis=1, keepdims=True)
  ed2 = jnp.sum(h2 * ad2_ref[...], axis=1, keepdims=True)
  ta_ref[...] = jnp.concatenate(
      [h2[:, 0:2], jnp.ones((bn, 1), jnp.float32),
       jnp.zeros((bn, 13), jnp.float32)], axis=1)
  tb_ref[...] = jnp.broadcast_to(es2, (bn, 16))
  tc_ref[...] = jnp.broadcast_to(ed2, (bn, 16))


def _tc_mid(acc, den, b1r, w2p, as2, ad2):
  grid = (NPAD // BN,)
  return pl.pallas_call(
      _tc_mid_body,
      grid=grid,
      in_specs=[
          pl.BlockSpec((NP, BN, 128), lambda i: (0, i, 0)),
          pl.BlockSpec((BN, 16), lambda i: (i, 0)),
          pl.BlockSpec((1, HC), lambda i: (0, 0)),
          pl.BlockSpec((HC, 128), lambda i: (0, 0)),
          pl.BlockSpec((1, 128), lambda i: (0, 0)),
          pl.BlockSpec((1, 128), lambda i: (0, 0)),
      ],
      out_specs=[
          pl.BlockSpec((BN, 16), lambda i: (i, 0)),
          pl.BlockSpec((BN, 16), lambda i: (i, 0)),
          pl.BlockSpec((BN, 16), lambda i: (i, 0)),
      ],
      out_shape=[
          jax.ShapeDtypeStruct((NPAD, 16), jnp.float32),
          jax.ShapeDtypeStruct((NPAD, 16), jnp.float32),
          jax.ShapeDtypeStruct((NPAD, 16), jnp.float32),
      ],
  )(acc, den, b1r, w2p, as2, ad2)


def _sc_l2_body(ta, tb, tc, src_h, dst_h, acc2_out,
                a2_s, srcb, dstb, gA, gB, gC, vb, zbd, gsem, ssem):
  c = lax.axis_index("c")
  s = lax.axis_index("s")
  row0 = s * ROWS_PER_TILE

  def zrow(i, _):
    zbd[i] = jnp.zeros((LANES,), jnp.float32)
    return 0
  lax.fori_loop(0, ZROWS, zrow, 0)
  for r in range(ROWS_PER_TILE // ZROWS):
    pltpu.sync_copy(zbd, a2_s.at[pl.ds(row0 + r * ZROWS, ZROWS)])
  plsc.subcore_barrier()

  wid = c * NSUB + s
  ept = E // (NCORES * NSUB)

  def issue(i, nb):
    off = pl.multiple_of(wid * ept + i * K2, 8)
    pltpu.sync_copy(src_h.at[pl.ds(off, K2)], srcb.at[nb])
    pltpu.sync_copy(dst_h.at[pl.ds(off, K2)], dstb.at[nb])
    pltpu.async_copy(ta.at[srcb.at[nb]], gA.at[nb], gsem.at[nb])
    pltpu.async_copy(tb.at[srcb.at[nb]], gB.at[nb], gsem.at[nb])
    pltpu.async_copy(tc.at[dstb.at[nb]], gC.at[nb], gsem.at[nb])

  def drain(ib):
    pltpu.make_async_copy(ta.at[srcb.at[ib]], gA.at[ib], gsem.at[ib]).wait()
    pltpu.make_async_copy(tb.at[srcb.at[ib]], gB.at[ib], gsem.at[ib]).wait()
    pltpu.make_async_copy(tc.at[dstb.at[ib]], gC.at[ib], gsem.at[ib]).wait()

  def drain_scat(ib):
    pltpu.make_async_copy(vb.at[ib], a2_s.at[dstb.at[ib]],
                          ssem.at[ib]).wait()

  issue(0, 0)

  def chunk(i, _):
    ib = lax.rem(i, 2)
    nb = 1 - ib

    @pl.when(i >= 1)
    def _():
      drain_scat(nb)

    @pl.when(i + 1 < NCH2)
    def _():
      issue(i + 1, nb)

    drain(ib)

    @plsc.parallel_loop(0, K2, unroll=8)
    def _(k):
      e = gB[ib, k] + gC[ib, k]
      ev = jnp.exp(jnp.maximum(e, 0.2 * e))
      vb[ib, k] = gA[ib, k] * ev
    pltpu.async_copy(vb.at[ib], a2_s.at[dstb.at[ib]], ssem.at[ib], add=True)
    return 0

  lax.fori_loop(0, NCH2, chunk, 0)
  drain_scat((NCH2 - 1) % 2)
  plsc.subcore_barrier()
  pltpu.sync_copy(a2_s.at[pl.ds(row0, ROWS_PER_TILE)],
                  acc2_out.at[c, pl.ds(row0, ROWS_PER_TILE)])


def _sc_l2(ta, tb, tc, src, dst):
  mesh = plsc.VectorSubcoreMesh(core_axis_name="c", subcore_axis_name="s",
                                num_cores=NCORES, num_subcores=NSUB)
  f = pl.kernel(
      _sc_l2_body,
      compiler_params=pltpu.CompilerParams(use_tc_tiling_on_sc=False),
      out_type=[jax.ShapeDtypeStruct((NCORES, NPAD, 16), jnp.float32)],
      mesh=mesh,
      scratch_types=[
          pltpu.VMEM_SHARED((NPAD, 16), jnp.float32),
          pltpu.VMEM((2, K2), jnp.int32),
          pltpu.VMEM((2, K2), jnp.int32),
          pltpu.VMEM((2, K2, 16), jnp.float32),
          pltpu.VMEM((2, K2, 16), jnp.float32),
          pltpu.VMEM((2, K2, 16), jnp.float32),
          pltpu.VMEM((2, K2, 16), jnp.float32),
          pltpu.VMEM((ZROWS, 16), jnp.float32),
          pltpu.SemaphoreType.DMA((2,)),
          pltpu.SemaphoreType.DMA((2,)),
      ],
  )
  return f(ta, tb, tc, src, dst)[0]


def _tc_fin_body(acc2_ref, b2_ref, out_ref):
  t = acc2_ref[0] + acc2_ref[1]
  d = t[:, 2:3] + 1e-16
  out_ref[...] = t / d + b2_ref[...]


def _tc_fin(acc2, b2p):
  grid = (NPAD // BN,)
  return pl.pallas_call(
      _tc_fin_body,
      grid=grid,
      in_specs=[
          pl.BlockSpec((NCORES, BN, 16), lambda i: (0, i, 0)),
          pl.BlockSpec((1, 16), lambda i: (0, 0)),
      ],
      out_specs=pl.BlockSpec((BN, 16), lambda i: (i, 0)),
      out_shape=jax.ShapeDtypeStruct((NPAD, 16), jnp.float32),
  )(acc2, b2p)


def kernel(x, edge_index, W1, a_src1, a_dst1, b1, W2, a_src2, a_dst2, b2):
  mask = jnp.repeat(jnp.eye(H, dtype=jnp.float32), C, axis=0)
  asrc = jnp.pad(mask * a_src1.reshape(HC, 1), ((0, 0), (0, 8)))
  adst = jnp.pad(mask * a_dst1.reshape(HC, 1), ((0, 0), (0, 8)))
  b1r = b1.reshape(1, HC)
  w2p = jnp.pad(W2, ((0, 0), (0, 126)))
  as2 = jnp.pad(a_src2.reshape(1, 2), ((0, 0), (0, 126)))
  ad2 = jnp.pad(a_dst2.reshape(1, 2), ((0, 0), (0, 126)))
  b2p = jnp.pad(b2, (0, 14)).reshape(1, 16)
  src = edge_index[0]
  dst = edge_index[1]
  xp = jnp.pad(x, ((0, NPAD - N), (0, 0)))

  h1t, es_t, ed_t = _tc_feats(xp, W1, asrc, adst)
  h1f = h1t.reshape(NP * NPAD, 128)
  acc, den = _sc_l1(h1f, es_t, ed_t, src, dst)
  ta, tb, tc = _tc_mid(acc, den, b1r, w2p, as2, ad2)
  acc2 = _sc_l2(ta, tb, tc, src, dst)
  res = _tc_fin(acc2, b2p)
  return res[:N, :2]


# speedup vs baseline: 1.2341x; 1.2341x over previous
import functools

import jax
import jax.numpy as jnp
from jax import lax
from jax.experimental import pallas as pl
from jax.experimental.pallas import tpu as pltpu
from jax.experimental.pallas import tpu_sc as plsc

N = 10000
NPAD = 10240
E = 160000
F_IN = 128
H = 8
C = 64
HC = H * C
NP = 4
NCORES = 2
NSUB = 16
LANES = 16

BN = 1024

K1 = 80
NCH1 = (E // NSUB) // K1
K2 = 40
NCH2 = (E // (NCORES * NSUB)) // K2

ROWS_PER_TILE = NPAD // NSUB
ZROWS = 64


def _tc_feats_body(x_ref, w1_ref, asrc_ref, adst_ref, h1t_ref, es_ref, ed_ref):
  h = jnp.dot(x_ref[...], w1_ref[...], preferred_element_type=jnp.float32)
  es_ref[...] = jnp.dot(h, asrc_ref[...], preferred_element_type=jnp.float32)
  ed_ref[...] = jnp.dot(h, adst_ref[...], preferred_element_type=jnp.float32)
  for p in range(NP):
    h1t_ref[p] = h[:, 128 * p:128 * (p + 1)]


def _tc_feats(x, w1, asrc, adst):
  grid = (NPAD // BN,)
  return pl.pallas_call(
      _tc_feats_body,
      grid=grid,
      in_specs=[
          pl.BlockSpec((BN, F_IN), lambda i: (i, 0)),
          pl.BlockSpec((F_IN, HC), lambda i: (0, 0)),
          pl.BlockSpec((HC, 16), lambda i: (0, 0)),
          pl.BlockSpec((HC, 16), lambda i: (0, 0)),
      ],
      out_specs=[
          pl.BlockSpec((NP, BN, 128), lambda i: (0, i, 0)),
          pl.BlockSpec((BN, 16), lambda i: (i, 0)),
          pl.BlockSpec((BN, 16), lambda i: (i, 0)),
      ],
      out_shape=[
          jax.ShapeDtypeStruct((NP, NPAD, 128), jnp.float32),
          jax.ShapeDtypeStruct((NPAD, 16), jnp.float32),
          jax.ShapeDtypeStruct((NPAD, 16), jnp.float32),
      ],
  )(x, w1, asrc, adst)


def _lane_take(vec, idx):
  return lax.gather(
      vec, idx[:, None],
      dimension_numbers=lax.GatherDimensionNumbers(
          offset_dims=(), collapsed_slice_dims=(0,), start_index_map=(0,)),
      slice_sizes=(1,),
      mode=lax.GatherScatterMode.PROMISE_IN_BOUNDS)


def _sc_l1_body(h1f, es_t, ed_t, ei_h,
                acc_out, den_out,
                acc_s, den_s, idxb, srcb2, g1, g2, rows, exb, zb, zbd,
                gsem, ssem, dsem):
  c = lax.axis_index("c")
  s = lax.axis_index("s")
  row0 = s * ROWS_PER_TILE

  def _zero_vec(ref, nrows, ncols):
    def zrow(i, _):
      for t in range(ncols // LANES):
        ref[i, pl.ds(t * LANES, LANES)] = jnp.zeros((LANES,), jnp.float32)
      return 0
    lax.fori_loop(0, nrows, zrow, 0)

  _zero_vec(zb, ZROWS, 128)
  _zero_vec(zbd, ZROWS, 16)

  def zero_acc():
    for r in range(ROWS_PER_TILE // ZROWS):
      pltpu.sync_copy(zb, acc_s.at[pl.ds(row0 + r * ZROWS, ZROWS)])

  zero_acc()
  for r in range(ROWS_PER_TILE // ZROWS):
    pltpu.sync_copy(zbd, den_s.at[pl.ds(row0 + r * ZROWS, ZROWS)])
  plsc.subcore_barrier()

  ept = E // NSUB

  def issue(i, nb, poff):
    off = pl.multiple_of(s * ept + i * K1, 8)
    pltpu.sync_copy(ei_h.at[:, pl.ds(off, K1)], idxb.at[nb])
    for t in range(K1 // LANES):
      srcb2[nb, pl.ds(t * LANES, LANES)] = (
          idxb[nb, 0, pl.ds(t * LANES, LANES)] + poff)
    pltpu.async_copy(es_t.at[idxb.at[nb, 0]], g1.at[nb], gsem.at[nb])
    pltpu.async_copy(ed_t.at[idxb.at[nb, 1]], g2.at[nb], gsem.at[nb])
    pltpu.async_copy(h1f.at[srcb2.at[nb]], rows.at[nb], gsem.at[nb])

  def drain_g12(ib):
    pltpu.make_async_copy(es_t.at[idxb.at[ib, 0]], g1.at[ib],
                          gsem.at[ib]).wait()
    pltpu.make_async_copy(ed_t.at[idxb.at[ib, 1]], g2.at[ib],
                          gsem.at[ib]).wait()

  def drain_rows(ib):
    pltpu.make_async_copy(h1f.at[srcb2.at[ib]], rows.at[ib],
                          gsem.at[ib]).wait()

  def drain_scat(ib, q):
    pltpu.make_async_copy(rows.at[ib], acc_s.at[idxb.at[ib, 1]],
                          ssem.at[ib]).wait()
    if q == 0:
      pltpu.make_async_copy(exb.at[ib], den_s.at[idxb.at[ib, 1]],
                            dsem.at[ib]).wait()

  for q in range(2):
    p = 2 * c + q
    hA = jnp.full((LANES,), 2 * p, jnp.int32)
    hB = jnp.full((LANES,), 2 * p + 1, jnp.int32)
    poff = p * NPAD

    issue(0, 0, poff)

    def chunk(i, _):
      ib = lax.rem(i, 2)
      nb = 1 - ib

      @pl.when(i >= 1)
      def _():
        drain_scat(nb, q)

      @pl.when(i + 1 < NCH1)
      def _():
        issue(i + 1, nb, poff)

      drain_g12(ib)

      @plsc.parallel_loop(0, K1, unroll=4)
      def _(k):
        e = g1[ib, k] + g2[ib, k]
        exb[ib, k] = jnp.exp(jnp.maximum(e, 0.2 * e))

      drain_rows(ib)

      @plsc.parallel_loop(0, K1, unroll=2)
      def _(k):
        ex = exb[ib, k]
        bA = _lane_take(ex, hA)
        bB = _lane_take(ex, hB)
        for j in range(8):
          b = bA if j < 4 else bB
          rows[ib, k, pl.ds(16 * j, LANES)] = (
              rows[ib, k, pl.ds(16 * j, LANES)] * b)
      pltpu.async_copy(rows.at[ib], acc_s.at[idxb.at[ib, 1]], ssem.at[ib],
                       add=True)
      if q == 0:
        pltpu.async_copy(exb.at[ib], den_s.at[idxb.at[ib, 1]], dsem.at[ib],
                         add=True)
      return 0

    lax.fori_loop(0, NCH1, chunk, 0)
    drain_scat((NCH1 - 1) % 2, q)
    plsc.subcore_barrier()
    pltpu.sync_copy(acc_s.at[pl.ds(row0, ROWS_PER_TILE)],
                    acc_out.at[p, pl.ds(row0, ROWS_PER_TILE)])
    if q == 0:
      @pl.when(c == 0)
      def _():
        pltpu.sync_copy(den_s.at[pl.ds(row0, ROWS_PER_TILE)],
                        den_out.at[pl.ds(row0, ROWS_PER_TILE)])
      zero_acc()
      plsc.subcore_barrier()


def _sc_l1(h1f, es_t, ed_t, ei):
  mesh = plsc.VectorSubcoreMesh(core_axis_name="c", subcore_axis_name="s",
                                num_cores=NCORES, num_subcores=NSUB)
  f = pl.kernel(
      _sc_l1_body,
      compiler_params=pltpu.CompilerParams(use_tc_tiling_on_sc=False),
      out_type=[
          jax.ShapeDtypeStruct((NP, NPAD, 128), jnp.float32),
          jax.ShapeDtypeStruct((NPAD, 16), jnp.float32),
      ],
      mesh=mesh,
      scratch_types=[
          pltpu.VMEM_SHARED((NPAD, 128), jnp.float32),
          pltpu.VMEM_SHARED((NPAD, 16), jnp.float32),
          pltpu.VMEM((2, 2, K1), jnp.int32),
          pltpu.VMEM((2, K1), jnp.int32),
          pltpu.VMEM((2, K1, 16), jnp.float32),
          pltpu.VMEM((2, K1, 16), jnp.float32),
          pltpu.VMEM((2, K1, 128), jnp.float32),
          pltpu.VMEM((2, K1, 16), jnp.float32),
          pltpu.VMEM((ZROWS, 128), jnp.float32),
          pltpu.VMEM((ZROWS, 16), jnp.float32),
          pltpu.SemaphoreType.DMA((2,)),
          pltpu.SemaphoreType.DMA((2,)),
          pltpu.SemaphoreType.DMA((2,)),
      ],
  )
  return f(h1f, es_t, ed_t, ei)


def _tc_mid_body(acc_ref, den_ref, b1_ref, w2_ref, as2_ref, ad2_ref,
                 ta_ref, tb_ref, tc_ref):
  den = den_ref[...]
  bn = den.shape[0]
  h2 = jnp.zeros((bn, 128), jnp.float32)
  for p in range(NP):
    dA = jnp.broadcast_to(den[:, 2 * p:2 * p + 1], (bn, 64))
    dB = jnp.broadcast_to(den[:, 2 * p + 1:2 * p + 2], (bn, 64))
    d128 = jnp.concatenate([dA, dB], axis=1)
    hp = acc_ref[p] / (d128 + 1e-16) + b1_ref[:, 128 * p:128 * (p + 1)]
    hp = jnp.where(hp > 0, hp, jnp.exp(hp) - 1.0)
    h2 = h2 + jnp.dot(hp, w2_ref[pl.ds(128 * p, 128), :],
                      preferred_element_type=jnp.float32)
  es2 = jnp.sum(h2 * as2_ref[...], axis=1, keepdims=True)
  ed2 = jnp.sum(h2 * ad2_ref[...], axis=1, keepdims=True)
  ta_ref[...] = jnp.concatenate(
      [h2[:, 0:2], jnp.ones((bn, 1), jnp.float32),
       jnp.zeros((bn, 13), jnp.float32)], axis=1)
  tb_ref[...] = jnp.broadcast_to(es2, (bn, 16))
  tc_ref[...] = jnp.broadcast_to(ed2, (bn, 16))


def _tc_mid(acc, den, b1r, w2p, as2, ad2):
  grid = (NPAD // BN,)
  return pl.pallas_call(
      _tc_mid_body,
      grid=grid,
      in_specs=[
          pl.BlockSpec((NP, BN, 128), lambda i: (0, i, 0)),
          pl.BlockSpec((BN, 16), lambda i: (i, 0)),
          pl.BlockSpec((1, HC), lambda i: (0, 0)),
          pl.BlockSpec((HC, 128), lambda i: (0, 0)),
          pl.BlockSpec((1, 128), lambda i: (0, 0)),
          pl.BlockSpec((1, 128), lambda i: (0, 0)),
      ],
      out_specs=[
          pl.BlockSpec((BN, 16), lambda i: (i, 0)),
          pl.BlockSpec((BN, 16), lambda i: (i, 0)),
          pl.BlockSpec((BN, 16), lambda i: (i, 0)),
      ],
      out_shape=[
          jax.ShapeDtypeStruct((NPAD, 16), jnp.float32),
          jax.ShapeDtypeStruct((NPAD, 16), jnp.float32),
          jax.ShapeDtypeStruct((NPAD, 16), jnp.float32),
      ],
  )(acc, den, b1r, w2p, as2, ad2)


def _sc_l2_body(ta, tb, tc, ei_h, acc2_out,
                a2_s, idxb, gA, gB, gC, vb, zbd, gsem, ssem):
  c = lax.axis_index("c")
  s = lax.axis_index("s")
  row0 = s * ROWS_PER_TILE

  def zrow(i, _):
    zbd[i] = jnp.zeros((LANES,), jnp.float32)
    return 0
  lax.fori_loop(0, ZROWS, zrow, 0)
  for r in range(ROWS_PER_TILE // ZROWS):
    pltpu.sync_copy(zbd, a2_s.at[pl.ds(row0 + r * ZROWS, ZROWS)])
  plsc.subcore_barrier()

  wid = c * NSUB + s
  ept = E // (NCORES * NSUB)

  def issue(i, nb):
    off = pl.multiple_of(wid * ept + i * K2, 8)
    pltpu.sync_copy(ei_h.at[:, pl.ds(off, K2)], idxb.at[nb])
    pltpu.async_copy(ta.at[idxb.at[nb, 0]], gA.at[nb], gsem.at[nb])
    pltpu.async_copy(tb.at[idxb.at[nb, 0]], gB.at[nb], gsem.at[nb])
    pltpu.async_copy(tc.at[idxb.at[nb, 1]], gC.at[nb], gsem.at[nb])

  def drain(ib):
    pltpu.make_async_copy(ta.at[idxb.at[ib, 0]], gA.at[ib],
                          gsem.at[ib]).wait()
    pltpu.make_async_copy(tb.at[idxb.at[ib, 0]], gB.at[ib],
                          gsem.at[ib]).wait()
    pltpu.make_async_copy(tc.at[idxb.at[ib, 1]], gC.at[ib],
                          gsem.at[ib]).wait()

  def drain_scat(ib):
    pltpu.make_async_copy(vb.at[ib], a2_s.at[idxb.at[ib, 1]],
                          ssem.at[ib]).wait()

  issue(0, 0)

  def chunk(i, _):
    ib = lax.rem(i, 2)
    nb = 1 - ib

    @pl.when(i >= 1)
    def _():
      drain_scat(nb)

    @pl.when(i + 1 < NCH2)
    def _():
      issue(i + 1, nb)

    drain(ib)

    @plsc.parallel_loop(0, K2, unroll=4)
    def _(k):
      e = gB[ib, k] + gC[ib, k]
      ev = jnp.exp(jnp.maximum(e, 0.2 * e))
      vb[ib, k] = gA[ib, k] * ev
    pltpu.async_copy(vb.at[ib], a2_s.at[idxb.at[ib, 1]], ssem.at[ib], add=True)
    return 0

  lax.fori_loop(0, NCH2, chunk, 0)
  drain_scat((NCH2 - 1) % 2)
  plsc.subcore_barrier()
  pltpu.sync_copy(a2_s.at[pl.ds(row0, ROWS_PER_TILE)],
                  acc2_out.at[c, pl.ds(row0, ROWS_PER_TILE)])


def _sc_l2(ta, tb, tc, ei):
  mesh = plsc.VectorSubcoreMesh(core_axis_name="c", subcore_axis_name="s",
                                num_cores=NCORES, num_subcores=NSUB)
  f = pl.kernel(
      _sc_l2_body,
      compiler_params=pltpu.CompilerParams(use_tc_tiling_on_sc=False),
      out_type=[jax.ShapeDtypeStruct((NCORES, NPAD, 16), jnp.float32)],
      mesh=mesh,
      scratch_types=[
          pltpu.VMEM_SHARED((NPAD, 16), jnp.float32),
          pltpu.VMEM((2, 2, K2), jnp.int32),
          pltpu.VMEM((2, K2, 16), jnp.float32),
          pltpu.VMEM((2, K2, 16), jnp.float32),
          pltpu.VMEM((2, K2, 16), jnp.float32),
          pltpu.VMEM((2, K2, 16), jnp.float32),
          pltpu.VMEM((ZROWS, 16), jnp.float32),
          pltpu.SemaphoreType.DMA((2,)),
          pltpu.SemaphoreType.DMA((2,)),
      ],
  )
  return f(ta, tb, tc, ei)[0]


def _tc_fin_body(acc2_ref, b2_ref, out_ref):
  t = acc2_ref[0] + acc2_ref[1]
  d = t[:, 2:3] + 1e-16
  out_ref[...] = t / d + b2_ref[...]


def _tc_fin(acc2, b2p):
  grid = (NPAD // BN,)
  return pl.pallas_call(
      _tc_fin_body,
      grid=grid,
      in_specs=[
          pl.BlockSpec((NCORES, BN, 16), lambda i: (0, i, 0)),
          pl.BlockSpec((1, 16), lambda i: (0, 0)),
      ],
      out_specs=pl.BlockSpec((BN, 16), lambda i: (i, 0)),
      out_shape=jax.ShapeDtypeStruct((NPAD, 16), jnp.float32),
  )(acc2, b2p)


def kernel(x, edge_index, W1, a_src1, a_dst1, b1, W2, a_src2, a_dst2, b2):
  mask = jnp.repeat(jnp.eye(H, dtype=jnp.float32), C, axis=0)
  asrc = jnp.pad(mask * a_src1.reshape(HC, 1), ((0, 0), (0, 8)))
  adst = jnp.pad(mask * a_dst1.reshape(HC, 1), ((0, 0), (0, 8)))
  b1r = b1.reshape(1, HC)
  w2p = jnp.pad(W2, ((0, 0), (0, 126)))
  as2 = jnp.pad(a_src2.reshape(1, 2), ((0, 0), (0, 126)))
  ad2 = jnp.pad(a_dst2.reshape(1, 2), ((0, 0), (0, 126)))
  b2p = jnp.pad(b2, (0, 14)).reshape(1, 16)
  xp = jnp.pad(x, ((0, NPAD - N), (0, 0)))

  h1t, es_t, ed_t = _tc_feats(xp, W1, asrc, adst)
  h1f = h1t.reshape(NP * NPAD, 128)
  acc, den = _sc_l1(h1f, es_t, ed_t, edge_index)
  ta, tb, tc = _tc_mid(acc, den, b1r, w2p, as2, ad2)
  acc2 = _sc_l2(ta, tb, tc, edge_index)
  res = _tc_fin(acc2, b2p)
  return res[:N, :2]
